# baseline (speedup 1.0000x reference)
import jax
import jax.numpy as jnp
from jax import lax
from jax.experimental import pallas as pl
from jax.experimental.pallas import tpu as pltpu
from jax.experimental.pallas import tpu_sc as plsc

EMBED = 1024
B, S = 4, 8192
TOKENS = B * S
NUM_CORES = 2
NUM_SUBCORES = 16
NSLICE = 8
CW = EMBED // NSLICE
NGRP = NUM_CORES * NUM_SUBCORES // NSLICE
TPT = TOKENS // NGRP
CHT = 128
NCHK = TPT // CHT


def _table_idx_body(tok_ref, mask_ref, w_ref, b_ref, emb_ref, table_ref, idx_ref):
    t2 = lax.broadcasted_iota(jnp.int32, (512, 8), 0) & 255
    sh = lax.broadcasted_iota(jnp.int32, (512, 8), 1)
    bits = ((t2 >> sh) & 1).astype(jnp.float32)
    occ = lax.dot_general(bits, w_ref[...], (((1,), (0,)), ((), ())),
                          preferred_element_type=jnp.float32)
    tcol = lax.broadcasted_iota(jnp.int32, (512, 1), 0)
    tmod = tcol & 255
    masked = tcol >= 256
    esel = jnp.where(tmod == 0, emb_ref[0:1, :],
                     jnp.where(tmod == 1, emb_ref[1:2, :], emb_ref[2:3, :]))
    full = occ + b_ref[...] + jnp.where(masked, esel, 0.0)
    for s in range(NSLICE):
        table_ref[s] = full[:, s * CW:(s + 1) * CW]
    idx_ref[...] = tok_ref[...] + 256 * mask_ref[...].astype(jnp.int32)


def _sc_gather_body(table_hbm, idx_hbm, out_hbm, tbl_v, idx_v, stg0, stg1,
                    wsem0, wsem1):
    cid = lax.axis_index("c")
    sid = lax.axis_index("s")
    sl = sid % NSLICE
    grp = (sid // NSLICE) * NUM_CORES + cid
    col0 = sl * CW
    tok0 = grp * TPT
    ld_t = pltpu.async_copy(table_hbm.at[sl], tbl_v, wsem0)
    ld_i = pltpu.async_copy(idx_hbm.at[pl.ds(tok0, TPT)],
                            idx_v.at[pl.ds(0, TPT)], wsem1)
    ld_t.wait()
    ld_i.wait()
    stgs = (stg0, stg1)
    wsems = (wsem0, wsem1)
    def super_step(i, carry):
        for b in range(2):
            c = 2 * i + b

            @pl.when(c >= 2)
            def _drain():
                pltpu.make_async_copy(
                    stgs[b],
                    out_hbm.at[pl.ds(0, CHT), pl.ds(col0, CW)],
                    wsems[b]).wait()

            stg = stgs[b]

            def tok_body(t):
                row = idx_v[pl.ds(c * CHT + t, 16)][0]
                for k in range(CW // 16):
                    stg[t, pl.ds(16 * k, 16)] = tbl_v[row, pl.ds(16 * k, 16)]

            plsc.parallel_loop(0, CHT, 1, unroll=4)(tok_body)
            pltpu.async_copy(
                stgs[b],
                out_hbm.at[pl.ds(tok0 + c * CHT, CHT), pl.ds(col0, CW)],
                wsems[b])
        return carry

    lax.fori_loop(0, NCHK // 2, super_step, 0)
    for b in range(2):
        pltpu.make_async_copy(
            stgs[b], out_hbm.at[pl.ds(0, CHT), pl.ds(col0, CW)],
            wsems[b]).wait()


@jax.jit
def kernel(token_ids, mask, W_occ, b_occ, emb_table):
    table, idx = pl.pallas_call(
        _table_idx_body,
        out_shape=(
            jax.ShapeDtypeStruct((NSLICE, 512, CW), jnp.float32),
            jax.ShapeDtypeStruct((B, S), jnp.int32),
        ),
    )(token_ids.astype(jnp.int32), mask, W_occ,
      b_occ.reshape(1, EMBED), emb_table)

    gather = pl.kernel(
        _sc_gather_body,
        out_type=jax.ShapeDtypeStruct((TOKENS, EMBED), jnp.float32),
        mesh=plsc.VectorSubcoreMesh(core_axis_name="c", subcore_axis_name="s"),
        compiler_params=pltpu.CompilerParams(needs_layout_passes=False),
        scratch_types=[
            pltpu.VMEM((512, CW), jnp.float32),
            pltpu.VMEM((TPT + 16,), jnp.int32),
            pltpu.VMEM((CHT, CW), jnp.float32),
            pltpu.VMEM((CHT, CW), jnp.float32),
            pltpu.SemaphoreType.DMA,
            pltpu.SemaphoreType.DMA,
        ],
    )
    out = gather(table, idx.reshape(TOKENS))
    return out.reshape(B, S, EMBED)


# speedup vs baseline: 1.0009x; 1.0009x over previous
import jax
import jax.numpy as jnp
from jax import lax
from jax.experimental import pallas as pl
from jax.experimental.pallas import tpu as pltpu
from jax.experimental.pallas import tpu_sc as plsc

EMBED = 1024
B, S = 4, 8192
TOKENS = B * S
NUM_CORES = 2
NUM_SUBCORES = 16
NSLICE = 8
CW = EMBED // NSLICE
NGRP = NUM_CORES * NUM_SUBCORES // NSLICE
TPT = TOKENS // NGRP
CHT = 128
NCHK = TPT // CHT


def _table_idx_body(tok_ref, mask_ref, w_ref, b_ref, emb_ref, table_ref, idx_ref):
    t2 = lax.broadcasted_iota(jnp.int32, (512, 8), 0) & 255
    sh = lax.broadcasted_iota(jnp.int32, (512, 8), 1)
    bits = ((t2 >> sh) & 1).astype(jnp.float32)
    occ = lax.dot_general(bits, w_ref[...], (((1,), (0,)), ((), ())),
                          preferred_element_type=jnp.float32)
    tcol = lax.broadcasted_iota(jnp.int32, (512, 1), 0)
    tmod = tcol & 255
    masked = tcol >= 256
    esel = jnp.where(tmod == 0, emb_ref[0:1, :],
                     jnp.where(tmod == 1, emb_ref[1:2, :], emb_ref[2:3, :]))
    full = occ + b_ref[...] + jnp.where(masked, esel, 0.0)
    for s in range(NSLICE):
        table_ref[s] = full[:, s * CW:(s + 1) * CW]
    idx_ref[...] = tok_ref[...] + 256 * mask_ref[...].astype(jnp.int32)


def _sc_gather_body(table_hbm, idx_hbm, out_hbm, tbl_v, idx_v, stg0, stg1,
                    wsem0, wsem1):
    cid = lax.axis_index("c")
    sid = lax.axis_index("s")
    sl = sid % NSLICE
    grp = (sid // NSLICE) * NUM_CORES + cid
    col0 = sl * CW
    tok0 = grp * TPT
    ld_t = pltpu.async_copy(table_hbm.at[sl], tbl_v, wsem0)
    ld_i = pltpu.async_copy(idx_hbm.at[pl.ds(tok0, TPT)],
                            idx_v.at[pl.ds(0, TPT)], wsem1)
    ld_t.wait()
    ld_i.wait()
    stgs = (stg0, stg1)
    wsems = (wsem0, wsem1)
    def super_step(i, carry):
        for b in range(2):
            c = 2 * i + b

            @pl.when(c >= 2)
            def _drain():
                pltpu.make_async_copy(
                    stgs[b],
                    out_hbm.at[pl.ds(0, CHT), pl.ds(col0, CW)],
                    wsems[b]).wait()

            stg = stgs[b]

            def tok_body(t):
                row = idx_v[pl.ds(c * CHT + t, 16)][0]
                for k in range(CW // 16):
                    stg[t, pl.ds(16 * k, 16)] = tbl_v[row, pl.ds(16 * k, 16)]

            plsc.parallel_loop(0, CHT, 1, unroll=2)(tok_body)
            pltpu.async_copy(
                stgs[b],
                out_hbm.at[pl.ds(tok0 + c * CHT, CHT), pl.ds(col0, CW)],
                wsems[b])
        return carry

    lax.fori_loop(0, NCHK // 2, super_step, 0)
    for b in range(2):
        pltpu.make_async_copy(
            stgs[b], out_hbm.at[pl.ds(0, CHT), pl.ds(col0, CW)],
            wsems[b]).wait()


@jax.jit
def kernel(token_ids, mask, W_occ, b_occ, emb_table):
    table, idx = pl.pallas_call(
        _table_idx_body,
        out_shape=(
            jax.ShapeDtypeStruct((NSLICE, 512, CW), jnp.float32),
            jax.ShapeDtypeStruct((B, S), jnp.int32),
        ),
    )(token_ids.astype(jnp.int32), mask, W_occ,
      b_occ.reshape(1, EMBED), emb_table)

    gather = pl.kernel(
        _sc_gather_body,
        out_type=jax.ShapeDtypeStruct((TOKENS, EMBED), jnp.float32),
        mesh=plsc.VectorSubcoreMesh(core_axis_name="c", subcore_axis_name="s"),
        compiler_params=pltpu.CompilerParams(needs_layout_passes=False),
        scratch_types=[
            pltpu.VMEM((512, CW), jnp.float32),
            pltpu.VMEM((TPT + 16,), jnp.int32),
            pltpu.VMEM((CHT, CW), jnp.float32),
            pltpu.VMEM((CHT, CW), jnp.float32),
            pltpu.SemaphoreType.DMA,
            pltpu.SemaphoreType.DMA,
        ],
    )
    out = gather(table, idx.reshape(TOKENS))
    return out.reshape(B, S, EMBED)
